# baseline (speedup 1.0000x reference)
import functools

import jax
import jax.numpy as jnp
from jax import lax
from jax.experimental import pallas as pl
from jax.experimental.pallas import tpu as pltpu
from jax.experimental.pallas import tpu_sc as plsc

_B = 16384
_CD = 64
_SD = 16
_NST = 51
_HID = 256
_OUT = 256
_NC = 2
_NS = 16
_NW = _NC * _NS
_NSPLIT = 4
_BS = _B // _NSPLIT
_BPW = _BS // _NW
_NCH = _BPW // 16


def _sc_gather(tbl3, cust_idx):
    mesh = plsc.VectorSubcoreMesh(core_axis_name="c", subcore_axis_name="s")

    @functools.partial(
        pl.kernel,
        mesh=mesh,
        compiler_params=pltpu.CompilerParams(needs_layout_passes=False),
        out_type=jax.ShapeDtypeStruct((_BS, _CD), jnp.float32),
        scratch_types=[
            pltpu.VMEM((_BPW,), jnp.int32),
            pltpu.VMEM((2, 8, 8, 256), jnp.float32),
            pltpu.VMEM((_BPW, _CD), jnp.float32),
            pltpu.SemaphoreType.DMA,
            pltpu.SemaphoreType.DMA,
        ],
    )
    def k(tbl, cidx, ce_out, idx_v, ring_v, rows_v, sem0, sem1):
        wid = lax.axis_index("s") * _NC + lax.axis_index("c")
        base = wid * _BPW
        pltpu.sync_copy(cidx.at[pl.ds(base, _BPW)], idx_v)
        lanes = lax.iota(jnp.int32, 16)
        sems = (sem0, sem1)

        def issue(c, buf):
            iv = idx_v[pl.ds(c * 16, 16)]
            j_all = lax.shift_left(lax.shift_right_logical(iv, 4), 4)
            for m in range(16):
                j = pl.multiple_of(j_all[m], 16)
                pltpu.async_copy(tbl.at[:, :, pl.ds(j, 16)],
                                 ring_v.at[buf, :, :, pl.ds(m * 16, 16)],
                                 sems[buf])

        def drain(buf):
            for m in range(16):
                pltpu.make_async_copy(
                    tbl.at[:, :, pl.ds(0, 16)],
                    ring_v.at[buf, :, :, pl.ds(m * 16, 16)], sems[buf]).wait()

        def extract(c, buf):
            iv = idx_v[pl.ds(c * 16, 16)]
            lo_vec = lanes * 16 + lax.bitwise_and(iv, 15)
            base_row = c * 16
            for m in range(16):
                lv = jnp.full((16,), lo_vec[m], jnp.int32)
                row = jnp.full((16,), base_row + m, jnp.int32)
                for kk in range(_CD // 16):
                    c_vec = lanes + kk * 16
                    diag = plsc.load_gather(
                        ring_v.at[buf],
                        [lax.shift_right_logical(c_vec, 3),
                         lax.bitwise_and(c_vec, 7), lv])
                    plsc.store_scatter(rows_v, [row, c_vec], diag)

        issue(0, 0)

        def body(t, carry):
            ca = 2 * t
            cb = 2 * t + 1
            issue(cb, 1)
            drain(0)
            extract(ca, 0)

            @pl.when(cb + 1 < _NCH)
            def _():
                issue(cb + 1, 0)

            drain(1)
            extract(cb, 1)
            return carry

        lax.fori_loop(0, _NCH // 2, body, 0)
        pltpu.sync_copy(rows_v, ce_out.at[pl.ds(base, _BPW)])

    return k(tbl3, cust_idx)


def _tc_mlp(ce, sid, feats, se_tbl, w1r, b1, w2, b2):
    blk = 2048
    grid = _BS // blk

    def body(ce_ref, sid_ref, f_ref, st_ref, w1_ref, b1_ref, w2_ref, b2_ref,
             o_ref):
        srow = lax.broadcasted_iota(jnp.int32, (1, 64), 1)
        onehot = (sid_ref[...] == srow).astype(jnp.float32)
        se = lax.dot_general(onehot, st_ref[...], (((1,), (0,)), ((), ())),
                             preferred_element_type=jnp.float32)
        x = jnp.concatenate([ce_ref[...], se, f_ref[...]], axis=1)
        h = lax.dot_general(x, w1_ref[...], (((1,), (1,)), ((), ())),
                            preferred_element_type=jnp.float32)
        h = jnp.maximum(h + b1_ref[...], 0.0)
        o = lax.dot_general(h, w2_ref[...], (((1,), (1,)), ((), ())),
                            preferred_element_type=jnp.float32)
        o = o + b2_ref[...]
        ss = jnp.sum(o * o, axis=1, keepdims=True)
        o_ref[...] = o / jnp.maximum(jnp.sqrt(ss), 1e-12)

    return pl.pallas_call(
        body,
        grid=(grid,),
        in_specs=[
            pl.BlockSpec((blk, _CD), lambda i: (i, 0)),
            pl.BlockSpec((blk, 1), lambda i: (i, 0)),
            pl.BlockSpec((blk, 48), lambda i: (i, 0)),
            pl.BlockSpec((64, _SD), lambda i: (0, 0)),
            pl.BlockSpec((_HID, 128), lambda i: (0, 0)),
            pl.BlockSpec((1, _HID), lambda i: (0, 0)),
            pl.BlockSpec((_OUT, _HID), lambda i: (0, 0)),
            pl.BlockSpec((1, _OUT), lambda i: (0, 0)),
        ],
        out_specs=pl.BlockSpec((blk, _OUT), lambda i: (i, 0)),
        out_shape=jax.ShapeDtypeStruct((_BS, _OUT), jnp.float32),
    )(ce, sid, feats, se_tbl, w1r, b1, w2, b2)


def kernel(customer_id, age, gender_onehot, state_id, is_student, total_spend,
           coupon_engagement, coupon_redemption_rate, avg_basket_size,
           customer_embed, state_embed, fc1_w, fc1_b, fc2_w, fc2_b):
    tbl3 = customer_embed.T.reshape(8, 8, 1000000)
    cid = customer_id.astype(jnp.int32)
    feats = jnp.concatenate([
        age[:, None], gender_onehot,
        is_student[:, None], total_spend[:, None], coupon_engagement[:, None],
        coupon_redemption_rate[:, None], avg_basket_size[:, None],
        jnp.zeros((_B, 39), jnp.float32),
    ], axis=1)
    w1r = jnp.concatenate([
        fc1_w[:, 0:64], fc1_w[:, 68:84], fc1_w[:, 64:68], fc1_w[:, 84:89],
        jnp.zeros((_HID, 39), jnp.float32),
    ], axis=1)
    se_tbl = jnp.concatenate([state_embed,
                              jnp.zeros((64 - _NST, _SD), jnp.float32)], axis=0)
    sid = state_id.astype(jnp.int32)[:, None]
    b1 = fc1_b[None, :]
    b2 = fc2_b[None, :]

    outs = []
    for s in range(_NSPLIT):
        lo = s * _BS
        ce_s = _sc_gather(tbl3, lax.slice(cid, (lo,), (lo + _BS,)))
        outs.append(_tc_mlp(
            ce_s,
            lax.slice(sid, (lo, 0), (lo + _BS, 1)),
            lax.slice(feats, (lo, 0), (lo + _BS, 48)),
            se_tbl, w1r, b1, w2=fc2_w, b2=b2))
    return jnp.concatenate(outs, axis=0)


# speedup vs baseline: 1.3033x; 1.3033x over previous
import functools

import jax
import jax.numpy as jnp
from jax import lax
from jax.experimental import pallas as pl
from jax.experimental.pallas import tpu as pltpu
from jax.experimental.pallas import tpu_sc as plsc

_B = 16384
_CD = 64
_SD = 16
_NST = 51
_HID = 256
_OUT = 256
_NC = 2
_NS = 16
_NW = _NC * _NS
_BPW = _B // _NW
_NCH = _BPW // 16


def _sc_gather(tbl3, cust_idx):
    mesh = plsc.VectorSubcoreMesh(core_axis_name="c", subcore_axis_name="s")

    @functools.partial(
        pl.kernel,
        mesh=mesh,
        compiler_params=pltpu.CompilerParams(needs_layout_passes=False),
        out_type=jax.ShapeDtypeStruct((_B, _CD), jnp.float32),
        scratch_types=[
            pltpu.VMEM((_BPW,), jnp.int32),
            pltpu.VMEM((2, 8, 8, 256), jnp.float32),
            pltpu.VMEM((_BPW, _CD), jnp.float32),
            pltpu.SemaphoreType.DMA,
            pltpu.SemaphoreType.DMA,
        ],
    )
    def k(tbl, cidx, ce_out, idx_v, ring_v, rows_v, sem0, sem1):
        wid = lax.axis_index("s") * _NC + lax.axis_index("c")
        base = wid * _BPW
        pltpu.sync_copy(cidx.at[pl.ds(base, _BPW)], idx_v)
        lanes = lax.iota(jnp.int32, 16)
        sems = (sem0, sem1)

        def issue(c, buf):
            iv = idx_v[pl.ds(c * 16, 16)]
            j_all = lax.shift_left(lax.shift_right_logical(iv, 4), 4)
            for m in range(16):
                j = pl.multiple_of(j_all[m], 16)
                pltpu.async_copy(tbl.at[:, :, pl.ds(j, 16)],
                                 ring_v.at[buf, :, :, pl.ds(m * 16, 16)],
                                 sems[buf])

        def drain(buf):
            for m in range(16):
                pltpu.make_async_copy(
                    tbl.at[:, :, pl.ds(0, 16)],
                    ring_v.at[buf, :, :, pl.ds(m * 16, 16)], sems[buf]).wait()

        def extract(c, buf):
            iv = idx_v[pl.ds(c * 16, 16)]
            lo_vec = lanes * 16 + lax.bitwise_and(iv, 15)
            base_row = c * 16
            for m in range(16):
                lv = jnp.full((16,), lo_vec[m], jnp.int32)
                row = jnp.full((16,), base_row + m, jnp.int32)
                for kk in range(_CD // 16):
                    c_vec = lanes + kk * 16
                    diag = plsc.load_gather(
                        ring_v.at[buf],
                        [lax.shift_right_logical(c_vec, 3),
                         lax.bitwise_and(c_vec, 7), lv])
                    plsc.store_scatter(rows_v, [row, c_vec], diag)

        issue(0, 0)

        def body(t, carry):
            ca = 2 * t
            cb = 2 * t + 1
            issue(cb, 1)
            drain(0)
            extract(ca, 0)

            @pl.when(cb + 1 < _NCH)
            def _():
                issue(cb + 1, 0)

            drain(1)
            extract(cb, 1)
            return carry

        lax.fori_loop(0, _NCH // 2, body, 0)
        pltpu.sync_copy(rows_v, ce_out.at[pl.ds(base, _BPW)])

    return k(tbl3, cust_idx)


def _tc_mlp(ce, sid, feats, se_tbl, w1r, b1, w2, b2):
    blk = 4096
    grid = _B // blk

    def body(ce_ref, sid_ref, f_ref, st_ref, w1_ref, b1_ref, w2_ref, b2_ref,
             o_ref):
        srow = lax.broadcasted_iota(jnp.int32, (1, 64), 1)
        onehot = (sid_ref[...] == srow).astype(jnp.float32)
        se = lax.dot_general(onehot, st_ref[...], (((1,), (0,)), ((), ())),
                             preferred_element_type=jnp.float32)
        x = jnp.concatenate([ce_ref[...], se, f_ref[...]], axis=1)
        h = lax.dot_general(x, w1_ref[...], (((1,), (1,)), ((), ())),
                            preferred_element_type=jnp.float32)
        h = jnp.maximum(h + b1_ref[...], 0.0)
        o = lax.dot_general(h, w2_ref[...], (((1,), (1,)), ((), ())),
                            preferred_element_type=jnp.float32)
        o = o + b2_ref[...]
        ss = jnp.sum(o * o, axis=1, keepdims=True)
        o_ref[...] = o / jnp.maximum(jnp.sqrt(ss), 1e-12)

    return pl.pallas_call(
        body,
        grid=(grid,),
        in_specs=[
            pl.BlockSpec((blk, _CD), lambda i: (i, 0)),
            pl.BlockSpec((blk, 1), lambda i: (i, 0)),
            pl.BlockSpec((blk, 48), lambda i: (i, 0)),
            pl.BlockSpec((64, _SD), lambda i: (0, 0)),
            pl.BlockSpec((_HID, 128), lambda i: (0, 0)),
            pl.BlockSpec((1, _HID), lambda i: (0, 0)),
            pl.BlockSpec((_OUT, _HID), lambda i: (0, 0)),
            pl.BlockSpec((1, _OUT), lambda i: (0, 0)),
        ],
        out_specs=pl.BlockSpec((blk, _OUT), lambda i: (i, 0)),
        out_shape=jax.ShapeDtypeStruct((_B, _OUT), jnp.float32),
    )(ce, sid, feats, se_tbl, w1r, b1, w2, b2)


def kernel(customer_id, age, gender_onehot, state_id, is_student, total_spend,
           coupon_engagement, coupon_redemption_rate, avg_basket_size,
           customer_embed, state_embed, fc1_w, fc1_b, fc2_w, fc2_b):
    tbl3 = customer_embed.T.reshape(8, 8, 1000000)
    cid = customer_id.astype(jnp.int32)
    feats = jnp.concatenate([
        age[:, None], gender_onehot,
        is_student[:, None], total_spend[:, None], coupon_engagement[:, None],
        coupon_redemption_rate[:, None], avg_basket_size[:, None],
        jnp.zeros((_B, 39), jnp.float32),
    ], axis=1)
    w1r = jnp.concatenate([
        fc1_w[:, 0:64], fc1_w[:, 68:84], fc1_w[:, 64:68], fc1_w[:, 84:89],
        jnp.zeros((_HID, 39), jnp.float32),
    ], axis=1)
    se_tbl = jnp.concatenate([state_embed,
                              jnp.zeros((64 - _NST, _SD), jnp.float32)], axis=0)
    ce = _sc_gather(tbl3, cid)
    return _tc_mlp(ce, state_id.astype(jnp.int32)[:, None], feats, se_tbl,
                   w1r, fc1_b[None, :], fc2_w, fc2_b[None, :])
